# baseline (speedup 1.0000x reference)
import functools

import jax
import jax.numpy as jnp
from jax import lax
from jax.experimental import pallas as pl
from jax.experimental.pallas import tpu as pltpu
from jax.experimental.pallas import tpu_sc as plsc

NUM_EMB = 100000
EMB_DIM = 64
BATCH = 16384
HIDDEN = 64

NC = 2
NS = 16
NW = NC * NS
IDX_W = 128
ROWS_PER_W = BATCH // (NW * IDX_W)

TBLK = 8192


def _build_body(emt_ref, edt_ref, eye_ref, out_ref):
    dims = (((0,), (0,)), ((), ()))
    tm = jax.lax.dot_general(emt_ref[...], eye_ref[...], dims,
                             preferred_element_type=jnp.float32)
    td = jax.lax.dot_general(edt_ref[...], eye_ref[...], dims,
                             preferred_element_type=jnp.float32)
    out_ref[...] = jnp.concatenate([tm, td], axis=-1)


def _tc_build_table(EmT, EdT):
    grid = ((NUM_EMB + TBLK - 1) // TBLK,)
    return pl.pallas_call(
        _build_body,
        grid=grid,
        in_specs=[
            pl.BlockSpec((EMB_DIM, TBLK), lambda i: (0, i)),
            pl.BlockSpec((EMB_DIM, TBLK), lambda i: (0, i)),
            pl.BlockSpec((EMB_DIM, EMB_DIM), lambda i: (0, 0)),
        ],
        out_specs=pl.BlockSpec((TBLK, 2 * EMB_DIM), lambda i: (i, 0)),
        out_shape=jax.ShapeDtypeStruct((NUM_EMB, 2 * EMB_DIM), jnp.float32),
    )(EmT, EdT, jnp.eye(EMB_DIM, dtype=jnp.float32))


def _sc_compose(sim_data, m_idx, d_idx):
    mesh = plsc.VectorSubcoreMesh(core_axis_name="c", subcore_axis_name="s")
    out_sh = jax.ShapeDtypeStruct((NW, ROWS_PER_W, IDX_W), jnp.int32)

    @functools.partial(
        pl.kernel,
        mesh=mesh,
        out_type=[out_sh, out_sh],
        scratch_types=[
            pltpu.VMEM((ROWS_PER_W, IDX_W), jnp.int32),
            pltpu.VMEM((ROWS_PER_W, IDX_W), jnp.int32),
            pltpu.VMEM((ROWS_PER_W, IDX_W), jnp.int32),
            pltpu.VMEM((ROWS_PER_W, IDX_W), jnp.int32),
            pltpu.SemaphoreType.DMA,
        ],
    )
    def compose_kernel(sim_hbm, midx_hbm, didx_hbm, outm_hbm, outd_hbm,
                       mi_v, di_v, sm_v, sd_v, sem):
        wid = lax.axis_index("s") * NC + lax.axis_index("c")
        pltpu.sync_copy(midx_hbm.at[wid], mi_v)
        pltpu.sync_copy(didx_hbm.at[wid], di_v)
        copies = []
        for j in range(ROWS_PER_W):
            copies.append(
                pltpu.async_copy(sim_hbm.at[mi_v.at[j]], sm_v.at[j], sem))
            copies.append(
                pltpu.async_copy(sim_hbm.at[di_v.at[j]], sd_v.at[j], sem))
        for c in copies:
            c.wait()
        pltpu.sync_copy(sm_v, outm_hbm.at[wid])
        pltpu.sync_copy(sd_v, outd_hbm.at[wid])

    return compose_kernel(sim_data, m_idx, d_idx)


def _sc_row_gather(table, sm, sd, base, nrows):
    mesh = plsc.VectorSubcoreMesh(core_axis_name="c", subcore_axis_name="s")
    out_sh = jax.ShapeDtypeStruct(
        (NW, nrows, IDX_W, 2 * EMB_DIM), jnp.float32)

    @functools.partial(
        pl.kernel,
        mesh=mesh,
        out_type=[out_sh, out_sh],
        scratch_types=[
            pltpu.VMEM((nrows, IDX_W), jnp.int32),
            pltpu.VMEM((nrows, IDX_W), jnp.int32),
            pltpu.VMEM((nrows, IDX_W, 2 * EMB_DIM), jnp.float32),
            pltpu.VMEM((nrows, IDX_W, 2 * EMB_DIM), jnp.float32),
            pltpu.SemaphoreType.DMA,
        ],
    )
    def gather_kernel(table_hbm, sm_hbm, sd_hbm, outm_hbm, outd_hbm,
                      sm_v, sd_v, mrows_v, drows_v, sem):
        wid = lax.axis_index("s") * NC + lax.axis_index("c")
        pltpu.sync_copy(sm_hbm.at[wid, pl.ds(base, nrows)], sm_v)
        pltpu.sync_copy(sd_hbm.at[wid, pl.ds(base, nrows)], sd_v)
        copies = []
        for j in range(nrows):
            copies.append(pltpu.async_copy(
                table_hbm.at[sm_v.at[j]], mrows_v.at[j], sem))
            copies.append(pltpu.async_copy(
                table_hbm.at[sd_v.at[j]], drows_v.at[j], sem))
        for c in copies:
            c.wait()
        pltpu.sync_copy(mrows_v, outm_hbm.at[wid])
        pltpu.sync_copy(drows_v, outd_hbm.at[wid])

    return gather_kernel(table, sm, sd)


def _mlp_body(m_ref, d_ref, w1m_ref, w1d_ref, b1_ref, w2_ref, b2_ref,
              out_ref):
    h = jnp.dot(m_ref[...], w1m_ref[...], preferred_element_type=jnp.float32)
    h = h + jnp.dot(d_ref[...], w1d_ref[...],
                    preferred_element_type=jnp.float32)
    h = jax.nn.relu(h + b1_ref[...])
    z = jnp.dot(h, w2_ref[...], preferred_element_type=jnp.float32)
    res = jax.nn.sigmoid(z + b2_ref[...])
    out_ref[...] = res.reshape(out_ref.shape)


def _tc_mlp(gm, gd, W1m, W1d, b1, W2, b2):
    n = gm.shape[0]
    blk = 4096
    grid = (n // blk,)
    return pl.pallas_call(
        _mlp_body,
        grid=grid,
        in_specs=[
            pl.BlockSpec((blk, 2 * EMB_DIM), lambda i: (i, 0)),
            pl.BlockSpec((blk, 2 * EMB_DIM), lambda i: (i, 0)),
            pl.BlockSpec((2 * EMB_DIM, HIDDEN), lambda i: (0, 0)),
            pl.BlockSpec((2 * EMB_DIM, HIDDEN), lambda i: (0, 0)),
            pl.BlockSpec((1, HIDDEN), lambda i: (0, 0)),
            pl.BlockSpec((HIDDEN, 1), lambda i: (0, 0)),
            pl.BlockSpec((1, 1), lambda i: (0, 0)),
        ],
        out_specs=pl.BlockSpec((blk // IDX_W, IDX_W), lambda i: (i, 0)),
        out_shape=jax.ShapeDtypeStruct((n // IDX_W, IDX_W), jnp.float32),
    )(gm, gd, W1m, W1d, b1, W2, b2)


def _half_view(col, half):
    lo = half * (BATCH // 2)
    return col[lo:lo + BATCH // 2].reshape(NW, ROWS_PER_W // 2, IDX_W)


def kernel(sim_data, train_data, Em_table, Ed_table, W1, b1, W2, b2):
    m_col = train_data[:, 0]
    d_col = train_data[:, 1]
    m_idx = jnp.concatenate([_half_view(m_col, 0), _half_view(m_col, 1)],
                            axis=1)
    d_idx = jnp.concatenate([_half_view(d_col, 0), _half_view(d_col, 1)],
                            axis=1)
    table = _tc_build_table(Em_table.T, Ed_table.T)
    sm, sd = _sc_compose(sim_data, m_idx, d_idx)
    zeros = jnp.zeros((EMB_DIM, HIDDEN), jnp.float32)
    W1m = jnp.concatenate([W1[:EMB_DIM], zeros], axis=0)
    W1d = jnp.concatenate([zeros, W1[EMB_DIM:]], axis=0)
    b1r = b1.reshape(1, HIDDEN)
    b2r = b2.reshape(1, 1)
    outs = []
    half_rows = ROWS_PER_W // 2
    for half in range(2):
        gm, gd = _sc_row_gather(table, sm, sd, half * half_rows, half_rows)
        gm = gm.reshape(BATCH // 2, 2 * EMB_DIM)
        gd = gd.reshape(BATCH // 2, 2 * EMB_DIM)
        outs.append(_tc_mlp(gm, gd, W1m, W1d, b1r, W2, b2r))
    return jnp.concatenate(outs, axis=0).reshape(BATCH)


# speedup vs baseline: 1.0477x; 1.0477x over previous
import functools

import jax
import jax.numpy as jnp
from jax import lax
from jax.experimental import pallas as pl
from jax.experimental.pallas import tpu as pltpu
from jax.experimental.pallas import tpu_sc as plsc

NUM_EMB = 100000
EMB_DIM = 64
BATCH = 16384
HIDDEN = 64

NC = 2
NS = 16
NW = NC * NS
IDX_W = 128
ROWS_PER_W = BATCH // (NW * IDX_W)

TBLK = 8192


def _build_body(emt_ref, edt_ref, eye_ref, out_ref):
    dims = (((0,), (0,)), ((), ()))
    tm = jax.lax.dot_general(emt_ref[...], eye_ref[...], dims,
                             preferred_element_type=jnp.float32)
    td = jax.lax.dot_general(edt_ref[...], eye_ref[...], dims,
                             preferred_element_type=jnp.float32)
    out_ref[...] = jnp.concatenate([tm, td], axis=-1)


def _tc_build_table(EmT, EdT):
    grid = ((NUM_EMB + TBLK - 1) // TBLK,)
    return pl.pallas_call(
        _build_body,
        grid=grid,
        in_specs=[
            pl.BlockSpec((EMB_DIM, TBLK), lambda i: (0, i)),
            pl.BlockSpec((EMB_DIM, TBLK), lambda i: (0, i)),
            pl.BlockSpec((EMB_DIM, EMB_DIM), lambda i: (0, 0)),
        ],
        out_specs=pl.BlockSpec((TBLK, 2 * EMB_DIM), lambda i: (i, 0)),
        out_shape=jax.ShapeDtypeStruct((NUM_EMB, 2 * EMB_DIM), jnp.float32),
    )(EmT, EdT, jnp.eye(EMB_DIM, dtype=jnp.float32))


def _sc_compose(sim_data, m_idx, d_idx):
    mesh = plsc.VectorSubcoreMesh(core_axis_name="c", subcore_axis_name="s")
    out_sh = jax.ShapeDtypeStruct((NW, ROWS_PER_W, IDX_W), jnp.int32)

    @functools.partial(
        pl.kernel,
        mesh=mesh,
        out_type=[out_sh, out_sh],
        scratch_types=[
            pltpu.VMEM((ROWS_PER_W, IDX_W), jnp.int32),
            pltpu.VMEM((ROWS_PER_W, IDX_W), jnp.int32),
            pltpu.VMEM((ROWS_PER_W, IDX_W), jnp.int32),
            pltpu.VMEM((ROWS_PER_W, IDX_W), jnp.int32),
            pltpu.SemaphoreType.DMA,
        ],
    )
    def compose_kernel(sim_hbm, midx_hbm, didx_hbm, outm_hbm, outd_hbm,
                       mi_v, di_v, sm_v, sd_v, sem):
        wid = lax.axis_index("s") * NC + lax.axis_index("c")
        pltpu.sync_copy(midx_hbm.at[wid], mi_v)
        pltpu.sync_copy(didx_hbm.at[wid], di_v)
        copies = []
        for j in range(ROWS_PER_W):
            copies.append(
                pltpu.async_copy(sim_hbm.at[mi_v.at[j]], sm_v.at[j], sem))
            copies.append(
                pltpu.async_copy(sim_hbm.at[di_v.at[j]], sd_v.at[j], sem))
        for c in copies:
            c.wait()
        pltpu.sync_copy(sm_v, outm_hbm.at[wid])
        pltpu.sync_copy(sd_v, outd_hbm.at[wid])

    return compose_kernel(sim_data, m_idx, d_idx)


def _sc_row_gather(table, sm, sd, base, nrows):
    mesh = plsc.VectorSubcoreMesh(core_axis_name="c", subcore_axis_name="s")
    out_sh = jax.ShapeDtypeStruct(
        (NW, nrows, IDX_W, 2 * EMB_DIM), jnp.float32)

    @functools.partial(
        pl.kernel,
        mesh=mesh,
        out_type=[out_sh, out_sh],
        scratch_types=[
            pltpu.VMEM((nrows, IDX_W), jnp.int32),
            pltpu.VMEM((nrows, IDX_W), jnp.int32),
            pltpu.VMEM((nrows, IDX_W, 2 * EMB_DIM), jnp.float32),
            pltpu.SemaphoreType.DMA,
        ],
    )
    def gather_kernel(table_hbm, sm_hbm, sd_hbm, outm_hbm, outd_hbm,
                      sm_v, sd_v, rows_v, sem):
        wid = lax.axis_index("s") * NC + lax.axis_index("c")
        pltpu.sync_copy(sm_hbm.at[wid, pl.ds(base, nrows)], sm_v)
        pltpu.sync_copy(sd_hbm.at[wid, pl.ds(base, nrows)], sd_v)
        copies = [pltpu.async_copy(table_hbm.at[sm_v.at[j]], rows_v.at[j],
                                   sem)
                  for j in range(nrows)]
        for c in copies:
            c.wait()
        pltpu.sync_copy(rows_v, outm_hbm.at[wid])
        copies = [pltpu.async_copy(table_hbm.at[sd_v.at[j]], rows_v.at[j],
                                   sem)
                  for j in range(nrows)]
        for c in copies:
            c.wait()
        pltpu.sync_copy(rows_v, outd_hbm.at[wid])

    return gather_kernel(table, sm, sd)


def _mlp_body(m_ref, d_ref, w1m_ref, w1d_ref, b1_ref, w2_ref, b2_ref,
              out_ref):
    h = jnp.dot(m_ref[...], w1m_ref[...], preferred_element_type=jnp.float32)
    h = h + jnp.dot(d_ref[...], w1d_ref[...],
                    preferred_element_type=jnp.float32)
    h = jax.nn.relu(h + b1_ref[...])
    z = jnp.dot(h, w2_ref[...], preferred_element_type=jnp.float32)
    res = jax.nn.sigmoid(z + b2_ref[...])
    out_ref[...] = res.reshape(out_ref.shape)


def _tc_mlp(gm, gd, W1m, W1d, b1, W2, b2):
    n = gm.shape[0]
    blk = 4096
    grid = (n // blk,)
    return pl.pallas_call(
        _mlp_body,
        grid=grid,
        in_specs=[
            pl.BlockSpec((blk, 2 * EMB_DIM), lambda i: (i, 0)),
            pl.BlockSpec((blk, 2 * EMB_DIM), lambda i: (i, 0)),
            pl.BlockSpec((2 * EMB_DIM, HIDDEN), lambda i: (0, 0)),
            pl.BlockSpec((2 * EMB_DIM, HIDDEN), lambda i: (0, 0)),
            pl.BlockSpec((1, HIDDEN), lambda i: (0, 0)),
            pl.BlockSpec((HIDDEN, 1), lambda i: (0, 0)),
            pl.BlockSpec((1, 1), lambda i: (0, 0)),
        ],
        out_specs=pl.BlockSpec((blk // IDX_W, IDX_W), lambda i: (i, 0)),
        out_shape=jax.ShapeDtypeStruct((n // IDX_W, IDX_W), jnp.float32),
    )(gm, gd, W1m, W1d, b1, W2, b2)


def kernel(sim_data, train_data, Em_table, Ed_table, W1, b1, W2, b2):
    m_idx = train_data[:, 0].reshape(NW, ROWS_PER_W, IDX_W)
    d_idx = train_data[:, 1].reshape(NW, ROWS_PER_W, IDX_W)
    table = _tc_build_table(Em_table.T, Ed_table.T)
    sm, sd = _sc_compose(sim_data, m_idx, d_idx)
    gm, gd = _sc_row_gather(table, sm, sd, 0, ROWS_PER_W)
    gm = gm.reshape(BATCH, 2 * EMB_DIM)
    gd = gd.reshape(BATCH, 2 * EMB_DIM)
    zeros = jnp.zeros((EMB_DIM, HIDDEN), jnp.float32)
    W1m = jnp.concatenate([W1[:EMB_DIM], zeros], axis=0)
    W1d = jnp.concatenate([zeros, W1[EMB_DIM:]], axis=0)
    out = _tc_mlp(gm, gd, W1m, W1d, b1.reshape(1, HIDDEN), W2,
                  b2.reshape(1, 1))
    return out.reshape(BATCH)
